# baseline (speedup 1.0000x reference)
import functools

import jax
import jax.numpy as jnp
from jax.experimental import pallas as pl
from jax.experimental.pallas import tpu as pltpu
from jax.experimental.pallas import tpu_sc as plsc

_K = 64


def _stage_roll(x, bit, keep_max, j):
    fwd = jnp.roll(x, j, axis=0)
    bwd = jnp.roll(x, -j, axis=0)
    partner = jnp.where(bit, fwd, bwd)
    mx = jnp.maximum(x, partner)
    mn = jnp.minimum(x, partner)
    return jnp.where(keep_max, mx, mn)


def _stage_static(x, j, k):
    n, c = x.shape
    if k >= 2 * n:
        v = x.reshape(-1, 2, j, c)
        a, b = v[:, 0], v[:, 1]
        return jnp.concatenate(
            [jnp.maximum(a, b)[:, None], jnp.minimum(a, b)[:, None]],
            axis=1).reshape(n, c)
    v = x.reshape(-1, 2, k // (2 * j), 2, j, c)
    a0, a1 = v[:, 0, :, 0], v[:, 0, :, 1]
    b0, b1 = v[:, 1, :, 0], v[:, 1, :, 1]
    na = jnp.concatenate(
        [jnp.minimum(a0, a1)[:, :, None], jnp.maximum(a0, a1)[:, :, None]],
        axis=2)
    nb = jnp.concatenate(
        [jnp.maximum(b0, b1)[:, :, None], jnp.minimum(b0, b1)[:, :, None]],
        axis=2)
    return jnp.concatenate([na[:, None], nb[:, None]], axis=1).reshape(n, c)


def _topk_body(x_ref, o_ref):
    x = x_ref[0]
    n, c = x.shape
    iota = jax.lax.broadcasted_iota(jnp.int32, (n, 1), 0)
    bit = {j: (iota & j) != 0 for j in (1, 2, 4)}
    km = {}
    for k in (2, 4, 8, 16, 32, 64):
        asc = (iota & k) == 0
        for j in (1, 2, 4):
            if j < k:
                km[(j, k)] = bit[j] == asc
    for j in (1, 2, 4):
        km[(j, 0)] = jnp.logical_not(bit[j])
    for k in (2, 4, 8, 16, 32, 64):
        j = k // 2
        while j:
            if j >= 8:
                x = _stage_static(x, j, k)
            else:
                x = _stage_roll(x, bit[j][:n], km[(j, k)][:n], j)
            j //= 2
    while n > _K:
        x = x.reshape(n // 128, 2, _K, c)
        x = jnp.maximum(x[:, 0], x[:, 1]).reshape(n // 2, c)
        n //= 2
        k = 64 if n > _K else 0
        for j in (32, 16, 8, 4, 2, 1):
            if j >= 8:
                x = _stage_static(x, j, k if k else 4 * n)
            else:
                x = _stage_roll(x, bit[j][:n], km[(j, k)][:n], j)
    o_ref[0] = x


def _sc_topk(x):
    b, s, d = x.shape
    nw = 32
    groups = b * d // 16
    gpw = groups // nw
    xt = jnp.transpose(x.reshape(b, s, d // 16, 16), (0, 2, 1, 3))
    xt = xt.reshape(groups * s * 16)
    gwords = s * 16
    owords = _K * 16
    mesh = plsc.VectorSubcoreMesh(core_axis_name="c", subcore_axis_name="s")

    def row(buf, i):
        return buf[pl.ds(i * 16, 16)]

    @functools.partial(
        pl.kernel, mesh=mesh,
        out_type=jax.ShapeDtypeStruct((groups * owords,), jnp.float32),
        scratch_types=[
            pltpu.VMEM((gwords,), jnp.float32),
            pltpu.SemaphoreType.DMA,
        ],
    )
    def sck(x_hbm, out_hbm, buf, sem):
        wid = jax.lax.axis_index("s") * 2 + jax.lax.axis_index("c")

        def group_body(g, carry):
            gid = wid * gpw + g
            pltpu.async_copy(
                x_hbm.at[pl.ds(gid * gwords, gwords)], buf, sem).wait()

            def blk_body(q, c2):
                base = q * 64
                v = [row(buf, base + i) for i in range(64)]
                for k in (2, 4, 8, 16, 32, 64):
                    j = k // 2
                    while j:
                        for i in range(64):
                            if (i & j) == 0:
                                up = (i & k) == 0
                                lo, hi = v[i], v[i + j]
                                mn = jnp.minimum(lo, hi)
                                mx = jnp.maximum(lo, hi)
                                v[i], v[i + j] = (mn, mx) if up else (mx, mn)
                        j //= 2
                for i in range(64):
                    buf[pl.ds((base + i) * 16, 16)] = v[i]
                return c2

            jax.lax.fori_loop(0, s // 64, blk_body, 0)

            m = s // 64
            while m > 1:
                half = m // 2
                last = half == 1

                def pair_body(p, c2, last=last):
                    ab = p * 128
                    v = [jnp.maximum(row(buf, ab + i), row(buf, ab + 127 - i))
                         for i in range(64)]
                    for j in (32, 16, 8, 4, 2, 1):
                        for i in range(64):
                            if (i & j) == 0:
                                lo, hi = v[i], v[i + j]
                                mn = jnp.minimum(lo, hi)
                                mx = jnp.maximum(lo, hi)
                                v[i], v[i + j] = (mx, mn) if last else (mn, mx)
                    for i in range(64):
                        buf[pl.ds((p * 64 + i) * 16, 16)] = v[i]
                    return c2

                jax.lax.fori_loop(0, half, pair_body, 0)
                m = half

            pltpu.sync_copy(buf.at[pl.ds(0, owords)],
                            out_hbm.at[pl.ds(gid * owords, owords)])
            return carry

        jax.lax.fori_loop(0, gpw, group_body, 0)

    out_t = sck(xt).reshape(b, d // 16, _K, 16)
    return jnp.transpose(out_t, (0, 2, 1, 3)).reshape(b, _K, d)


def kernel(x):
    return _sc_topk(x)


def _tc_kernel(x):
    b, s, d = x.shape
    c = 512
    return pl.pallas_call(
        _topk_body,
        grid=(b, d // c),
        in_specs=[pl.BlockSpec((1, s, c), lambda i, j: (i, 0, j))],
        out_specs=pl.BlockSpec((1, _K, c), lambda i, j: (i, 0, j)),
        out_shape=jax.ShapeDtypeStruct((b, _K, d), x.dtype),
        compiler_params=pltpu.CompilerParams(
            dimension_semantics=("parallel", "parallel")),
    )(x)


# speedup vs baseline: 1.1625x; 1.1625x over previous
import functools

import jax
import jax.numpy as jnp
from jax.experimental import pallas as pl
from jax.experimental.pallas import tpu as pltpu
from jax.experimental.pallas import tpu_sc as plsc

_K = 64


def _stage_roll(x, bit, keep_max, j):
    fwd = jnp.roll(x, j, axis=0)
    bwd = jnp.roll(x, -j, axis=0)
    partner = jnp.where(bit, fwd, bwd)
    mx = jnp.maximum(x, partner)
    mn = jnp.minimum(x, partner)
    return jnp.where(keep_max, mx, mn)


def _stage_static(x, j, k):
    n, c = x.shape
    if k >= 2 * n:
        v = x.reshape(-1, 2, j, c)
        a, b = v[:, 0], v[:, 1]
        return jnp.concatenate(
            [jnp.maximum(a, b)[:, None], jnp.minimum(a, b)[:, None]],
            axis=1).reshape(n, c)
    v = x.reshape(-1, 2, k // (2 * j), 2, j, c)
    a0, a1 = v[:, 0, :, 0], v[:, 0, :, 1]
    b0, b1 = v[:, 1, :, 0], v[:, 1, :, 1]
    na = jnp.concatenate(
        [jnp.minimum(a0, a1)[:, :, None], jnp.maximum(a0, a1)[:, :, None]],
        axis=2)
    nb = jnp.concatenate(
        [jnp.maximum(b0, b1)[:, :, None], jnp.minimum(b0, b1)[:, :, None]],
        axis=2)
    return jnp.concatenate([na[:, None], nb[:, None]], axis=1).reshape(n, c)


def _topk_body(x_ref, o_ref):
    x = x_ref[0]
    n, c = x.shape
    iota = jax.lax.broadcasted_iota(jnp.int32, (n, 1), 0)
    bit = {j: (iota & j) != 0 for j in (1, 2, 4)}
    km = {}
    for k in (2, 4, 8, 16, 32, 64):
        asc = (iota & k) == 0
        for j in (1, 2, 4):
            if j < k:
                km[(j, k)] = bit[j] == asc
    for j in (1, 2, 4):
        km[(j, 0)] = jnp.logical_not(bit[j])
    for k in (2, 4, 8, 16, 32, 64):
        j = k // 2
        while j:
            if j >= 8:
                x = _stage_static(x, j, k)
            else:
                x = _stage_roll(x, bit[j][:n], km[(j, k)][:n], j)
            j //= 2
    while n > _K:
        x = x.reshape(n // 128, 2, _K, c)
        x = jnp.maximum(x[:, 0], x[:, 1]).reshape(n // 2, c)
        n //= 2
        k = 64 if n > _K else 0
        for j in (32, 16, 8, 4, 2, 1):
            if j >= 8:
                x = _stage_static(x, j, k if k else 4 * n)
            else:
                x = _stage_roll(x, bit[j][:n], km[(j, k)][:n], j)
    o_ref[0] = x


def _sc_topk(x):
    b, s, d = x.shape
    nw = 32
    groups = b * d // 16
    gpw = groups // nw
    xt = jnp.transpose(x.reshape(b, s, d // 16, 16), (0, 2, 1, 3))
    xt = xt.reshape(groups * s * 16)
    gwords = s * 16
    owords = _K * 16
    mesh = plsc.VectorSubcoreMesh(core_axis_name="c", subcore_axis_name="s")

    def row(buf, i):
        return buf[pl.ds(i * 16, 16)]

    @functools.partial(
        pl.kernel, mesh=mesh,
        out_type=jax.ShapeDtypeStruct((groups * owords,), jnp.float32),
        scratch_types=[
            pltpu.VMEM((gwords,), jnp.float32),
            pltpu.SemaphoreType.DMA,
        ],
    )
    def sck(x_hbm, out_hbm, buf, sem):
        wid = jax.lax.axis_index("s") * 2 + jax.lax.axis_index("c")

        def group_body(g, carry):
            gid = wid * gpw + g
            pltpu.async_copy(
                x_hbm.at[pl.ds(gid * gwords, gwords)], buf, sem).wait()

            def blk_body(q, c2):
                base = q * 64
                v = [row(buf, base + i) for i in range(64)]
                for k in (2, 4, 8, 16, 32, 64):
                    j = k // 2
                    while j:
                        for i in range(64):
                            if (i & j) == 0:
                                up = (i & k) == 0
                                lo, hi = v[i], v[i + j]
                                mn = jnp.minimum(lo, hi)
                                mx = jnp.maximum(lo, hi)
                                v[i], v[i + j] = (mn, mx) if up else (mx, mn)
                        j //= 2
                for i in range(64):
                    buf[pl.ds((base + i) * 16, 16)] = v[i]
                return c2

            jax.lax.fori_loop(0, s // 64, blk_body, 0)

            m = s // 64
            while m > 1:
                half = m // 2
                last = half == 1

                def pair_body(p, c2, last=last):
                    ab = p * 128
                    v = [jnp.maximum(row(buf, ab + i), row(buf, ab + 127 - i))
                         for i in range(64)]
                    for j in (32, 16, 8, 4, 2, 1):
                        for i in range(64):
                            if (i & j) == 0:
                                lo, hi = v[i], v[i + j]
                                mn = jnp.minimum(lo, hi)
                                mx = jnp.maximum(lo, hi)
                                v[i], v[i + j] = (mx, mn) if last else (mn, mx)
                    for i in range(64):
                        buf[pl.ds((p * 64 + i) * 16, 16)] = v[i]
                    return c2

                jax.lax.fori_loop(0, half, pair_body, 0)
                m = half

            pltpu.sync_copy(buf.at[pl.ds(0, owords)],
                            out_hbm.at[pl.ds(gid * owords, owords)])
            return carry

        jax.lax.fori_loop(0, gpw, group_body, 0)

    out_t = sck(xt).reshape(b, d // 16, _K, 16)
    return jnp.transpose(out_t, (0, 2, 1, 3)).reshape(b, _K, d)


def _tc_kernel(x, dtc):
    b, s, d = x.shape
    c = min(512, dtc)
    return pl.pallas_call(
        _topk_body,
        grid=(b, dtc // c),
        in_specs=[pl.BlockSpec((1, s, c), lambda i, j: (i, 0, j))],
        out_specs=pl.BlockSpec((1, _K, c), lambda i, j: (i, 0, j)),
        out_shape=jax.ShapeDtypeStruct((b, _K, dtc), x.dtype),
        compiler_params=pltpu.CompilerParams(
            dimension_semantics=("parallel", "parallel")),
    )(x)


def kernel(x):
    b, s, d = x.shape
    dsc = 512
    sc_out = _sc_topk(x[:, :, d - dsc:])
    tc_out = _tc_kernel(x, d - dsc)
    return jnp.concatenate([tc_out, sc_out], axis=2)


# speedup vs baseline: 1.2192x; 1.0487x over previous
import functools

import jax
import jax.numpy as jnp
from jax.experimental import pallas as pl
from jax.experimental.pallas import tpu as pltpu
from jax.experimental.pallas import tpu_sc as plsc

_K = 64


def _stage_roll(x, bit, keep_max, j):
    fwd = jnp.roll(x, j, axis=0)
    bwd = jnp.roll(x, -j, axis=0)
    partner = jnp.where(bit, fwd, bwd)
    mx = jnp.maximum(x, partner)
    mn = jnp.minimum(x, partner)
    return jnp.where(keep_max, mx, mn)


def _stage_static(x, j, k):
    n, c = x.shape
    if k >= 2 * n:
        v = x.reshape(-1, 2, j, c)
        a, b = v[:, 0], v[:, 1]
        return jnp.concatenate(
            [jnp.maximum(a, b)[:, None], jnp.minimum(a, b)[:, None]],
            axis=1).reshape(n, c)
    v = x.reshape(-1, 2, k // (2 * j), 2, j, c)
    a0, a1 = v[:, 0, :, 0], v[:, 0, :, 1]
    b0, b1 = v[:, 1, :, 0], v[:, 1, :, 1]
    na = jnp.concatenate(
        [jnp.minimum(a0, a1)[:, :, None], jnp.maximum(a0, a1)[:, :, None]],
        axis=2)
    nb = jnp.concatenate(
        [jnp.maximum(b0, b1)[:, :, None], jnp.minimum(b0, b1)[:, :, None]],
        axis=2)
    return jnp.concatenate([na[:, None], nb[:, None]], axis=1).reshape(n, c)


def _topk_body(x_ref, o_ref):
    x = x_ref[0]
    n, c = x.shape
    iota = jax.lax.broadcasted_iota(jnp.int32, (n, 1), 0)
    bit = {j: (iota & j) != 0 for j in (1, 2, 4)}
    km = {}
    for k in (2, 4, 8, 16, 32, 64):
        asc = (iota & k) == 0
        for j in (1, 2, 4):
            if j < k:
                km[(j, k)] = bit[j] == asc
    for j in (1, 2, 4):
        km[(j, 0)] = jnp.logical_not(bit[j])
    for k in (2, 4, 8, 16, 32, 64):
        j = k // 2
        while j:
            if j >= 8:
                x = _stage_static(x, j, k)
            else:
                x = _stage_roll(x, bit[j][:n], km[(j, k)][:n], j)
            j //= 2
    while n > _K:
        x = x.reshape(n // 128, 2, _K, c)
        x = jnp.maximum(x[:, 0], x[:, 1]).reshape(n // 2, c)
        n //= 2
        k = 64 if n > _K else 0
        for j in (32, 16, 8, 4, 2, 1):
            if j >= 8:
                x = _stage_static(x, j, k if k else 4 * n)
            else:
                x = _stage_roll(x, bit[j][:n], km[(j, k)][:n], j)
    o_ref[0] = x


def _sc_topk(x):
    b, s, d = x.shape
    nw = 32
    groups = b * d // 16
    gpw = groups // nw
    xt = jnp.transpose(x.reshape(b, s, d // 16, 16), (0, 2, 1, 3))
    xt = xt.reshape(groups * s * 16)
    gwords = s * 16
    owords = _K * 16
    mesh = plsc.VectorSubcoreMesh(core_axis_name="c", subcore_axis_name="s")

    def row(buf, i):
        return buf[pl.ds(i * 16, 16)]

    @functools.partial(
        pl.kernel, mesh=mesh,
        out_type=jax.ShapeDtypeStruct((groups * owords,), jnp.float32),
        scratch_types=[
            pltpu.VMEM((gwords,), jnp.float32),
            pltpu.SemaphoreType.DMA,
        ],
    )
    def sck(x_hbm, out_hbm, buf, sem):
        wid = jax.lax.axis_index("s") * 2 + jax.lax.axis_index("c")

        def group_body(g, carry):
            gid = wid * gpw + g
            pltpu.async_copy(
                x_hbm.at[pl.ds(gid * gwords, gwords)], buf, sem).wait()

            def blk_body(q, c2):
                base = q * 64
                v = [row(buf, base + i) for i in range(64)]
                for k in (2, 4, 8, 16, 32, 64):
                    j = k // 2
                    while j:
                        for i in range(64):
                            if (i & j) == 0:
                                up = (i & k) == 0
                                lo, hi = v[i], v[i + j]
                                mn = jnp.minimum(lo, hi)
                                mx = jnp.maximum(lo, hi)
                                v[i], v[i + j] = (mn, mx) if up else (mx, mn)
                        j //= 2
                for i in range(64):
                    buf[pl.ds((base + i) * 16, 16)] = v[i]
                return c2

            jax.lax.fori_loop(0, s // 64, blk_body, 0)

            m = s // 64
            while m > 1:
                half = m // 2
                last = half == 1

                def pair_body(p, c2, last=last):
                    ab = p * 128
                    v = [jnp.maximum(row(buf, ab + i), row(buf, ab + 127 - i))
                         for i in range(64)]
                    for j in (32, 16, 8, 4, 2, 1):
                        for i in range(64):
                            if (i & j) == 0:
                                lo, hi = v[i], v[i + j]
                                mn = jnp.minimum(lo, hi)
                                mx = jnp.maximum(lo, hi)
                                v[i], v[i + j] = (mx, mn) if last else (mn, mx)
                    for i in range(64):
                        buf[pl.ds((p * 64 + i) * 16, 16)] = v[i]
                    return c2

                jax.lax.fori_loop(0, half, pair_body, 0)
                m = half

            pltpu.sync_copy(buf.at[pl.ds(0, owords)],
                            out_hbm.at[pl.ds(gid * owords, owords)])
            return carry

        jax.lax.fori_loop(0, gpw, group_body, 0)

    out_t = sck(xt).reshape(b, d // 16, _K, 16)
    return jnp.transpose(out_t, (0, 2, 1, 3)).reshape(b, _K, d)


def _tc_kernel(x, dtc):
    b, s, d = x.shape
    c = min(512, dtc)
    return pl.pallas_call(
        _topk_body,
        grid=(b, dtc // c),
        in_specs=[pl.BlockSpec((1, s, c), lambda i, j: (i, 0, j))],
        out_specs=pl.BlockSpec((1, _K, c), lambda i, j: (i, 0, j)),
        out_shape=jax.ShapeDtypeStruct((b, _K, dtc), x.dtype),
        compiler_params=pltpu.CompilerParams(
            dimension_semantics=("parallel", "parallel")),
    )(x)


def kernel(x):
    b, s, d = x.shape
    dsc = 768
    sc_out = _sc_topk(x[:, :, d - dsc:])
    tc_out = _tc_kernel(x, d - dsc)
    return jnp.concatenate([tc_out, sc_out], axis=2)
